# baseline (speedup 1.0000x reference)
import jax
import jax.numpy as jnp
from jax import lax
from jax.experimental import pallas as pl
from jax.experimental.pallas import tpu as pltpu
from jax.experimental.pallas import tpu_sc as plsc

N = 10000
E = 320000
D = 128
T = 4
STEPS = 8

NC = 2
NS = 16
NW = NC * NS
EPW = E // NW
CH = 128
G = 13
NGROUPS = 6
CTAIL = EPW - NGROUPS * G * CH
ZR = 624
ZTAIL = N - NS * ZR

_HI = jax.lax.Precision.HIGHEST


def _trans_body(h_ref, w_ref, b_ref, o_ref):
    h = h_ref[...]
    w = w_ref[0]
    o_ref[0] = lax.dot_general(h, w, (((1,), (1,)), ((), ())),
                               precision=_HI) + b_ref[0]


BNT = 1000
_trans_call = pl.pallas_call(
    _trans_body,
    grid=(T, N // BNT),
    in_specs=[
        pl.BlockSpec((BNT, D), lambda t, n: (n, 0)),
        pl.BlockSpec((1, D, D), lambda t, n: (t, 0, 0)),
        pl.BlockSpec((1, 1, D), lambda t, n: (t, 0, 0)),
    ],
    out_specs=pl.BlockSpec((1, BNT, D), lambda t, n: (t, n, 0)),
    out_shape=jax.ShapeDtypeStruct((T, N, D), jnp.float32),
)


def _edge_body(trans_hbm, gidx_hbm, dst_hbm, gidxt_hbm, dstt_hbm, zrows_hbm,
               out_hbm, gidx_v, didx_v, tidx_v, tdid_v, rows_a, rows_b, rows_t,
               acc_sh, sem_a, sem_b, sem_i0, sem_i1):
    c = lax.axis_index("c")
    s = lax.axis_index("s")
    w = c * NS + s
    pltpu.sync_copy(zrows_hbm, acc_sh.at[pl.ds(s * ZR, ZR)])

    @pl.when(s == NS - 1)
    def _():
        pltpu.sync_copy(zrows_hbm.at[pl.ds(0, ZTAIL)],
                        acc_sh.at[pl.ds(NS * ZR, ZTAIL)])

    pltpu.sync_copy(gidx_hbm.at[w, 0], gidx_v.at[0])
    pltpu.sync_copy(dst_hbm.at[w, 0], didx_v.at[0])
    pltpu.sync_copy(gidxt_hbm.at[w], tidx_v)
    pltpu.sync_copy(dstt_hbm.at[w], tdid_v)
    plsc.subcore_barrier()

    def group(g, carry):
        p = g & 1
        @pl.when(g < NGROUPS - 1)
        def _():
            pltpu.async_copy(gidx_hbm.at[w, g + 1], gidx_v.at[1 - p], sem_i1)
            pltpu.async_copy(dst_hbm.at[w, g + 1], didx_v.at[1 - p], sem_i0)

        @pl.when(g > 0)
        def _():
            pltpu.make_async_copy(gidx_hbm.at[w, g], gidx_v.at[p], sem_i1).wait()
            pltpu.make_async_copy(dst_hbm.at[w, g], didx_v.at[p], sem_i0).wait()

        pltpu.async_copy(trans_hbm.at[gidx_v.at[p, 0]], rows_a, sem_a)

        def body(j, carry2):
            pltpu.async_copy(trans_hbm.at[gidx_v.at[p, 2 * j + 1]], rows_b, sem_b)
            pltpu.make_async_copy(trans_hbm.at[gidx_v.at[p, 2 * j]],
                                  rows_a, sem_a).wait()
            pltpu.sync_copy(rows_a, acc_sh.at[didx_v.at[p, 2 * j]], add=True)
            pltpu.async_copy(trans_hbm.at[gidx_v.at[p, 2 * j + 2]], rows_a, sem_a)
            pltpu.make_async_copy(trans_hbm.at[gidx_v.at[p, 2 * j + 1]],
                                  rows_b, sem_b).wait()
            pltpu.sync_copy(rows_b, acc_sh.at[didx_v.at[p, 2 * j + 1]], add=True)
            return carry2

        lax.fori_loop(0, (G - 1) // 2, body, 0)
        pltpu.make_async_copy(trans_hbm.at[gidx_v.at[p, G - 1]],
                              rows_a, sem_a).wait()
        pltpu.sync_copy(rows_a, acc_sh.at[didx_v.at[p, G - 1]], add=True)
        return carry

    lax.fori_loop(0, NGROUPS, group, 0)
    pltpu.async_copy(trans_hbm.at[tidx_v.at[0]], rows_t, sem_a).wait()
    pltpu.sync_copy(rows_t, acc_sh.at[tdid_v.at[0]], add=True)
    plsc.subcore_barrier()
    pltpu.sync_copy(acc_sh.at[pl.ds(s * ZR, ZR)],
                    out_hbm.at[c, pl.ds(s * ZR, ZR)])

    @pl.when(s == NS - 1)
    def _():
        pltpu.sync_copy(acc_sh.at[pl.ds(NS * ZR, ZTAIL)],
                        out_hbm.at[c, pl.ds(NS * ZR, ZTAIL)])


_edge_call = pl.kernel(
    _edge_body,
    mesh=plsc.VectorSubcoreMesh(core_axis_name="c", subcore_axis_name="s"),
    out_type=jax.ShapeDtypeStruct((NC, N, D), jnp.float32),
    scratch_types=[
        pltpu.VMEM((2, G, CH), jnp.int32),
        pltpu.VMEM((2, G, CH), jnp.int32),
        pltpu.VMEM((1, CTAIL), jnp.int32),
        pltpu.VMEM((1, CTAIL), jnp.int32),
        pltpu.VMEM((CH, D), jnp.float32),
        pltpu.VMEM((CH, D), jnp.float32),
        pltpu.VMEM((CTAIL, D), jnp.float32),
        pltpu.VMEM_SHARED((N, D), jnp.float32),
        pltpu.SemaphoreType.DMA,
        pltpu.SemaphoreType.DMA,
        pltpu.SemaphoreType.DMA,
        pltpu.SemaphoreType.DMA,
    ],
)


def _gru_update(a_ref, h_ref, wih_ref, whh_ref, bih_ref, bhh_ref):
    a = a_ref[0] + a_ref[1]
    h = h_ref[...]
    gi = lax.dot_general(a, wih_ref[...], (((1,), (1,)), ((), ())),
                         precision=_HI) + bih_ref[...]
    gh = lax.dot_general(h, whh_ref[...], (((1,), (1,)), ((), ())),
                         precision=_HI) + bhh_ref[...]
    r = jax.nn.sigmoid(gi[:, :D] + gh[:, :D])
    z = jax.nn.sigmoid(gi[:, D:2 * D] + gh[:, D:2 * D])
    n = jnp.tanh(gi[:, 2 * D:] + r * gh[:, 2 * D:])
    return (1.0 - z) * n + z * h


def _gru_body(a_ref, h_ref, wih_ref, whh_ref, bih_ref, bhh_ref, o_ref):
    o_ref[...] = _gru_update(a_ref, h_ref, wih_ref, whh_ref, bih_ref, bhh_ref)


BNG = 1000
_gru_call = pl.pallas_call(
    _gru_body,
    grid=(N // BNG,),
    in_specs=[
        pl.BlockSpec((NC, BNG, D), lambda n: (0, n, 0)),
        pl.BlockSpec((BNG, D), lambda n: (n, 0)),
        pl.BlockSpec((3 * D, D), lambda n: (0, 0)),
        pl.BlockSpec((3 * D, D), lambda n: (0, 0)),
        pl.BlockSpec((1, 3 * D), lambda n: (0, 0)),
        pl.BlockSpec((1, 3 * D), lambda n: (0, 0)),
    ],
    out_specs=pl.BlockSpec((BNG, D), lambda n: (n, 0)),
    out_shape=jax.ShapeDtypeStruct((N, D), jnp.float32),
)


def _pool_body(h_ref, wc_ref, bc_ref, o_ref, acc_ref):
    @pl.when(pl.program_id(0) == 0)
    def _():
        acc_ref[...] = jnp.zeros_like(acc_ref)

    acc_ref[...] += jnp.sum(h_ref[...], axis=0, keepdims=True)

    @pl.when(pl.program_id(0) == (N // BNG) - 1)
    def _():
        logit = jnp.sum(acc_ref[...] * wc_ref[...]) + bc_ref[0, 0]
        o_ref[...] = jnp.full((1, 1), jax.nn.sigmoid(logit), jnp.float32)


_pool_call = pl.pallas_call(
    _pool_body,
    grid=(N // BNG,),
    in_specs=[
        pl.BlockSpec((BNG, D), lambda n: (n, 0)),
        pl.BlockSpec((1, D), lambda n: (0, 0)),
        pl.BlockSpec((1, 1), lambda n: (0, 0)),
    ],
    out_specs=pl.BlockSpec((1, 1), lambda n: (0, 0)),
    out_shape=jax.ShapeDtypeStruct((1, 1), jnp.float32),
    scratch_shapes=[pltpu.VMEM((1, D), jnp.float32)],
)


def kernel(x, edge_index, edge_types, W_e, b_e, w_ih, w_hh, b_ih, b_hh,
           W_cls, b_cls):
    src = edge_index[0]
    dst = edge_index[1]
    gflat = (edge_types * N + src).astype(jnp.int32).reshape(NW, EPW)
    dflat = dst.astype(jnp.int32).reshape(NW, EPW)
    nmain = NGROUPS * G * CH
    gidx = gflat[:, :nmain].reshape(NW, NGROUPS, G, CH)
    dst2 = dflat[:, :nmain].reshape(NW, NGROUPS, G, CH)
    gidxt = gflat[:, nmain:].reshape(NW, 1, CTAIL)
    dstt = dflat[:, nmain:].reshape(NW, 1, CTAIL)
    zrows = jnp.zeros((ZR, D), jnp.float32)
    bih2 = b_ih.reshape(1, 3 * D)
    bhh2 = b_hh.reshape(1, 3 * D)

    be3 = b_e.reshape(T, 1, D)
    h = x
    for _ in range(STEPS):
        trans = _trans_call(h, W_e, be3).reshape(T * N, D)
        parts = _edge_call(trans, gidx, dst2, gidxt, dstt, zrows)
        h = _gru_call(parts, h, w_ih, w_hh, bih2, bhh2)
    out = _pool_call(h, W_cls.reshape(1, D), b_cls.reshape(1, 1))
    return out[0, 0]


# speedup vs baseline: 1.0818x; 1.0818x over previous
import jax
import jax.numpy as jnp
from jax import lax
from jax.experimental import pallas as pl
from jax.experimental.pallas import tpu as pltpu
from jax.experimental.pallas import tpu_sc as plsc

N = 10000
E = 320000
D = 128
T = 4
STEPS = 8

NC = 2
NS = 16
NW = NC * NS
EPW = E // NW
CH = 128
G = 13
NGROUPS = 6
CTAIL = EPW - NGROUPS * G * CH
ZR = 624
ZTAIL = N - NS * ZR

_HI = jax.lax.Precision.HIGHEST


def _trans_body(h_ref, w_ref, b_ref, o_ref):
    o_ref[...] = lax.dot_general(h_ref[...], w_ref[...],
                                 (((1,), (0,)), ((), ())),
                                 precision=_HI) + b_ref[...]


BNT = 2000
_trans_call = pl.pallas_call(
    _trans_body,
    grid=(N // BNT,),
    in_specs=[
        pl.BlockSpec((BNT, D), lambda n: (n, 0)),
        pl.BlockSpec((D, T * D), lambda n: (0, 0)),
        pl.BlockSpec((1, T * D), lambda n: (0, 0)),
    ],
    out_specs=pl.BlockSpec((BNT, T * D), lambda n: (n, 0)),
    out_shape=jax.ShapeDtypeStruct((N, T * D), jnp.float32),
)


def _edge_body(trans_hbm, gidx_hbm, dst_hbm, gidxt_hbm, dstt_hbm, zrows_hbm,
               out_hbm, gidx_v, didx_v, tidx_v, tdid_v, rows_a, rows_b, rows_t,
               acc_sh, sem_a, sem_b, sem_i0, sem_i1):
    c = lax.axis_index("c")
    s = lax.axis_index("s")
    w = c * NS + s
    pltpu.sync_copy(zrows_hbm, acc_sh.at[pl.ds(s * ZR, ZR)])

    @pl.when(s == NS - 1)
    def _():
        pltpu.sync_copy(zrows_hbm.at[pl.ds(0, ZTAIL)],
                        acc_sh.at[pl.ds(NS * ZR, ZTAIL)])

    pltpu.sync_copy(gidx_hbm.at[w, 0], gidx_v.at[0])
    pltpu.sync_copy(dst_hbm.at[w, 0], didx_v.at[0])
    pltpu.sync_copy(gidxt_hbm.at[w], tidx_v)
    pltpu.sync_copy(dstt_hbm.at[w], tdid_v)
    plsc.subcore_barrier()

    def group(g, carry):
        p = g & 1
        @pl.when(g < NGROUPS - 1)
        def _():
            pltpu.async_copy(gidx_hbm.at[w, g + 1], gidx_v.at[1 - p], sem_i1)
            pltpu.async_copy(dst_hbm.at[w, g + 1], didx_v.at[1 - p], sem_i0)

        @pl.when(g > 0)
        def _():
            pltpu.make_async_copy(gidx_hbm.at[w, g], gidx_v.at[p], sem_i1).wait()
            pltpu.make_async_copy(dst_hbm.at[w, g], didx_v.at[p], sem_i0).wait()

        pltpu.async_copy(trans_hbm.at[gidx_v.at[p, 0]], rows_a, sem_a)

        def body(j, carry2):
            pltpu.async_copy(trans_hbm.at[gidx_v.at[p, 2 * j + 1]], rows_b, sem_b)
            pltpu.make_async_copy(trans_hbm.at[gidx_v.at[p, 2 * j]],
                                  rows_a, sem_a).wait()
            pltpu.sync_copy(rows_a, acc_sh.at[didx_v.at[p, 2 * j]], add=True)
            pltpu.async_copy(trans_hbm.at[gidx_v.at[p, 2 * j + 2]], rows_a, sem_a)
            pltpu.make_async_copy(trans_hbm.at[gidx_v.at[p, 2 * j + 1]],
                                  rows_b, sem_b).wait()
            pltpu.sync_copy(rows_b, acc_sh.at[didx_v.at[p, 2 * j + 1]], add=True)
            return carry2

        lax.fori_loop(0, (G - 1) // 2, body, 0)
        pltpu.make_async_copy(trans_hbm.at[gidx_v.at[p, G - 1]],
                              rows_a, sem_a).wait()
        pltpu.sync_copy(rows_a, acc_sh.at[didx_v.at[p, G - 1]], add=True)
        return carry

    lax.fori_loop(0, NGROUPS, group, 0)
    pltpu.async_copy(trans_hbm.at[tidx_v.at[0]], rows_t, sem_a).wait()
    pltpu.sync_copy(rows_t, acc_sh.at[tdid_v.at[0]], add=True)
    plsc.subcore_barrier()
    pltpu.sync_copy(acc_sh.at[pl.ds(s * ZR, ZR)],
                    out_hbm.at[c, pl.ds(s * ZR, ZR)])

    @pl.when(s == NS - 1)
    def _():
        pltpu.sync_copy(acc_sh.at[pl.ds(NS * ZR, ZTAIL)],
                        out_hbm.at[c, pl.ds(NS * ZR, ZTAIL)])


_edge_call = pl.kernel(
    _edge_body,
    mesh=plsc.VectorSubcoreMesh(core_axis_name="c", subcore_axis_name="s"),
    out_type=jax.ShapeDtypeStruct((NC, N, D), jnp.float32),
    scratch_types=[
        pltpu.VMEM((2, G, CH), jnp.int32),
        pltpu.VMEM((2, G, CH), jnp.int32),
        pltpu.VMEM((1, CTAIL), jnp.int32),
        pltpu.VMEM((1, CTAIL), jnp.int32),
        pltpu.VMEM((CH, D), jnp.float32),
        pltpu.VMEM((CH, D), jnp.float32),
        pltpu.VMEM((CTAIL, D), jnp.float32),
        pltpu.VMEM_SHARED((N, D), jnp.float32),
        pltpu.SemaphoreType.DMA,
        pltpu.SemaphoreType.DMA,
        pltpu.SemaphoreType.DMA,
        pltpu.SemaphoreType.DMA,
    ],
)


def _gru_update(a_ref, h_ref, wih_ref, whh_ref, bih_ref, bhh_ref):
    a = a_ref[0] + a_ref[1]
    h = h_ref[...]
    gi = lax.dot_general(a, wih_ref[...], (((1,), (1,)), ((), ())),
                         precision=_HI) + bih_ref[...]
    gh = lax.dot_general(h, whh_ref[...], (((1,), (1,)), ((), ())),
                         precision=_HI) + bhh_ref[...]
    r = jax.nn.sigmoid(gi[:, :D] + gh[:, :D])
    z = jax.nn.sigmoid(gi[:, D:2 * D] + gh[:, D:2 * D])
    n = jnp.tanh(gi[:, 2 * D:] + r * gh[:, 2 * D:])
    return (1.0 - z) * n + z * h


def _gru_body(a_ref, h_ref, wih_ref, whh_ref, bih_ref, bhh_ref, o_ref):
    o_ref[...] = _gru_update(a_ref, h_ref, wih_ref, whh_ref, bih_ref, bhh_ref)


BNG = 2000
_gru_call = pl.pallas_call(
    _gru_body,
    grid=(N // BNG,),
    in_specs=[
        pl.BlockSpec((NC, BNG, D), lambda n: (0, n, 0)),
        pl.BlockSpec((BNG, D), lambda n: (n, 0)),
        pl.BlockSpec((3 * D, D), lambda n: (0, 0)),
        pl.BlockSpec((3 * D, D), lambda n: (0, 0)),
        pl.BlockSpec((1, 3 * D), lambda n: (0, 0)),
        pl.BlockSpec((1, 3 * D), lambda n: (0, 0)),
    ],
    out_specs=pl.BlockSpec((BNG, D), lambda n: (n, 0)),
    out_shape=jax.ShapeDtypeStruct((N, D), jnp.float32),
)


def _pool_body(h_ref, wc_ref, bc_ref, o_ref, acc_ref):
    @pl.when(pl.program_id(0) == 0)
    def _():
        acc_ref[...] = jnp.zeros_like(acc_ref)

    acc_ref[...] += jnp.sum(h_ref[...], axis=0, keepdims=True)

    @pl.when(pl.program_id(0) == (N // BNG) - 1)
    def _():
        logit = jnp.sum(acc_ref[...] * wc_ref[...]) + bc_ref[0, 0]
        o_ref[...] = jnp.full((1, 1), jax.nn.sigmoid(logit), jnp.float32)


_pool_call = pl.pallas_call(
    _pool_body,
    grid=(N // BNG,),
    in_specs=[
        pl.BlockSpec((BNG, D), lambda n: (n, 0)),
        pl.BlockSpec((1, D), lambda n: (0, 0)),
        pl.BlockSpec((1, 1), lambda n: (0, 0)),
    ],
    out_specs=pl.BlockSpec((1, 1), lambda n: (0, 0)),
    out_shape=jax.ShapeDtypeStruct((1, 1), jnp.float32),
    scratch_shapes=[pltpu.VMEM((1, D), jnp.float32)],
)


def kernel(x, edge_index, edge_types, W_e, b_e, w_ih, w_hh, b_ih, b_hh,
           W_cls, b_cls):
    src = edge_index[0]
    dst = edge_index[1]
    gflat = (src * T + edge_types).astype(jnp.int32).reshape(NW, EPW)
    dflat = dst.astype(jnp.int32).reshape(NW, EPW)
    nmain = NGROUPS * G * CH
    gidx = gflat[:, :nmain].reshape(NW, NGROUPS, G, CH)
    dst2 = dflat[:, :nmain].reshape(NW, NGROUPS, G, CH)
    gidxt = gflat[:, nmain:].reshape(NW, 1, CTAIL)
    dstt = dflat[:, nmain:].reshape(NW, 1, CTAIL)
    zrows = jnp.zeros((ZR, D), jnp.float32)
    bih2 = b_ih.reshape(1, 3 * D)
    bhh2 = b_hh.reshape(1, 3 * D)

    w_pack = jnp.transpose(W_e, (2, 0, 1)).reshape(D, T * D)
    b_pack = b_e.reshape(1, T * D)
    h = x
    for _ in range(STEPS):
        trans = _trans_call(h, w_pack, b_pack).reshape(N * T, D)
        parts = _edge_call(trans, gidx, dst2, gidxt, dstt, zrows)
        h = _gru_call(parts, h, w_ih, w_hh, bih2, bhh2)
    out = _pool_call(h, W_cls.reshape(1, D), b_cls.reshape(1, 1))
    return out[0, 0]


# speedup vs baseline: 1.2078x; 1.1165x over previous
import jax
import jax.numpy as jnp
from jax import lax
from jax.experimental import pallas as pl
from jax.experimental.pallas import tpu as pltpu
from jax.experimental.pallas import tpu_sc as plsc

N = 10000
E = 320000
D = 128
T = 4
STEPS = 8

NC = 2
NS = 16
NW = NC * NS
EPW = E // NW
CH = 128
G = 13
NGROUPS = 6
CTAIL = EPW - NGROUPS * G * CH
ZR = 624
ZTAIL = N - NS * ZR

_HI = jax.lax.Precision.HIGHEST


def _trans_body(h_ref, w_ref, b_ref, o_ref):
    packed = lax.dot_general(h_ref[...], w_ref[...], (((1,), (0,)), ((), ())),
                             precision=_HI) + b_ref[...]
    for t in range(T):
        o_ref[t] = packed[:, t * D:(t + 1) * D]


BNT = 2000
_trans_call = pl.pallas_call(
    _trans_body,
    grid=(N // BNT,),
    in_specs=[
        pl.BlockSpec((BNT, D), lambda n: (n, 0)),
        pl.BlockSpec((D, T * D), lambda n: (0, 0)),
        pl.BlockSpec((1, T * D), lambda n: (0, 0)),
    ],
    out_specs=pl.BlockSpec((T, BNT, D), lambda n: (0, n, 0)),
    out_shape=jax.ShapeDtypeStruct((T, N, D), jnp.float32),
)


def _edge_body(trans_hbm, gidx_hbm, dst_hbm, gidxt_hbm, dstt_hbm, zrows_hbm,
               out_hbm, gidx_v, didx_v, tidx_v, tdid_v, rows_a, rows_b, rows_t,
               acc_sh, sem_a, sem_b, sem_i0, sem_i1):
    c = lax.axis_index("c")
    s = lax.axis_index("s")
    w = c * NS + s
    pltpu.sync_copy(zrows_hbm, acc_sh.at[pl.ds(s * ZR, ZR)])

    @pl.when(s == NS - 1)
    def _():
        pltpu.sync_copy(zrows_hbm.at[pl.ds(0, ZTAIL)],
                        acc_sh.at[pl.ds(NS * ZR, ZTAIL)])

    pltpu.sync_copy(gidx_hbm.at[w, 0], gidx_v.at[0])
    pltpu.sync_copy(dst_hbm.at[w, 0], didx_v.at[0])
    pltpu.sync_copy(gidxt_hbm.at[w], tidx_v)
    pltpu.sync_copy(dstt_hbm.at[w], tdid_v)
    plsc.subcore_barrier()

    def group(g, carry):
        p = g & 1
        @pl.when(g < NGROUPS - 1)
        def _():
            pltpu.async_copy(gidx_hbm.at[w, g + 1], gidx_v.at[1 - p], sem_i1)
            pltpu.async_copy(dst_hbm.at[w, g + 1], didx_v.at[1 - p], sem_i0)

        @pl.when(g > 0)
        def _():
            pltpu.make_async_copy(gidx_hbm.at[w, g], gidx_v.at[p], sem_i1).wait()
            pltpu.make_async_copy(dst_hbm.at[w, g], didx_v.at[p], sem_i0).wait()

        pltpu.async_copy(trans_hbm.at[gidx_v.at[p, 0]], rows_a, sem_a)

        def body(j, carry2):
            pltpu.async_copy(trans_hbm.at[gidx_v.at[p, 2 * j + 1]], rows_b, sem_b)
            pltpu.make_async_copy(trans_hbm.at[gidx_v.at[p, 2 * j]],
                                  rows_a, sem_a).wait()
            pltpu.sync_copy(rows_a, acc_sh.at[didx_v.at[p, 2 * j]], add=True)
            pltpu.async_copy(trans_hbm.at[gidx_v.at[p, 2 * j + 2]], rows_a, sem_a)
            pltpu.make_async_copy(trans_hbm.at[gidx_v.at[p, 2 * j + 1]],
                                  rows_b, sem_b).wait()
            pltpu.sync_copy(rows_b, acc_sh.at[didx_v.at[p, 2 * j + 1]], add=True)
            return carry2

        lax.fori_loop(0, (G - 1) // 2, body, 0)
        pltpu.make_async_copy(trans_hbm.at[gidx_v.at[p, G - 1]],
                              rows_a, sem_a).wait()
        pltpu.sync_copy(rows_a, acc_sh.at[didx_v.at[p, G - 1]], add=True)
        return carry

    lax.fori_loop(0, NGROUPS, group, 0)
    pltpu.async_copy(trans_hbm.at[tidx_v.at[0]], rows_t, sem_a).wait()
    pltpu.sync_copy(rows_t, acc_sh.at[tdid_v.at[0]], add=True)
    plsc.subcore_barrier()
    pltpu.sync_copy(acc_sh.at[pl.ds(s * ZR, ZR)],
                    out_hbm.at[c, pl.ds(s * ZR, ZR)])

    @pl.when(s == NS - 1)
    def _():
        pltpu.sync_copy(acc_sh.at[pl.ds(NS * ZR, ZTAIL)],
                        out_hbm.at[c, pl.ds(NS * ZR, ZTAIL)])


_edge_call = pl.kernel(
    _edge_body,
    mesh=plsc.VectorSubcoreMesh(core_axis_name="c", subcore_axis_name="s"),
    out_type=jax.ShapeDtypeStruct((NC, N, D), jnp.float32),
    scratch_types=[
        pltpu.VMEM((2, G, CH), jnp.int32),
        pltpu.VMEM((2, G, CH), jnp.int32),
        pltpu.VMEM((1, CTAIL), jnp.int32),
        pltpu.VMEM((1, CTAIL), jnp.int32),
        pltpu.VMEM((CH, D), jnp.float32),
        pltpu.VMEM((CH, D), jnp.float32),
        pltpu.VMEM((CTAIL, D), jnp.float32),
        pltpu.VMEM_SHARED((N, D), jnp.float32),
        pltpu.SemaphoreType.DMA,
        pltpu.SemaphoreType.DMA,
        pltpu.SemaphoreType.DMA,
        pltpu.SemaphoreType.DMA,
    ],
)


def _gru_update(a_ref, h_ref, wih_ref, whh_ref, bih_ref, bhh_ref):
    a = a_ref[0] + a_ref[1]
    h = h_ref[...]
    gi = lax.dot_general(a, wih_ref[...], (((1,), (1,)), ((), ())),
                         precision=_HI) + bih_ref[...]
    gh = lax.dot_general(h, whh_ref[...], (((1,), (1,)), ((), ())),
                         precision=_HI) + bhh_ref[...]
    r = jax.nn.sigmoid(gi[:, :D] + gh[:, :D])
    z = jax.nn.sigmoid(gi[:, D:2 * D] + gh[:, D:2 * D])
    n = jnp.tanh(gi[:, 2 * D:] + r * gh[:, 2 * D:])
    return (1.0 - z) * n + z * h


def _gru_body(a_ref, h_ref, wih_ref, whh_ref, bih_ref, bhh_ref, o_ref):
    o_ref[...] = _gru_update(a_ref, h_ref, wih_ref, whh_ref, bih_ref, bhh_ref)


BNG = 2000
_gru_call = pl.pallas_call(
    _gru_body,
    grid=(N // BNG,),
    in_specs=[
        pl.BlockSpec((NC, BNG, D), lambda n: (0, n, 0)),
        pl.BlockSpec((BNG, D), lambda n: (n, 0)),
        pl.BlockSpec((3 * D, D), lambda n: (0, 0)),
        pl.BlockSpec((3 * D, D), lambda n: (0, 0)),
        pl.BlockSpec((1, 3 * D), lambda n: (0, 0)),
        pl.BlockSpec((1, 3 * D), lambda n: (0, 0)),
    ],
    out_specs=pl.BlockSpec((BNG, D), lambda n: (n, 0)),
    out_shape=jax.ShapeDtypeStruct((N, D), jnp.float32),
)


def _pool_body(h_ref, wc_ref, bc_ref, o_ref, acc_ref):
    @pl.when(pl.program_id(0) == 0)
    def _():
        acc_ref[...] = jnp.zeros_like(acc_ref)

    acc_ref[...] += jnp.sum(h_ref[...], axis=0, keepdims=True)

    @pl.when(pl.program_id(0) == (N // BNG) - 1)
    def _():
        logit = jnp.sum(acc_ref[...] * wc_ref[...]) + bc_ref[0, 0]
        o_ref[...] = jnp.full((1, 1), jax.nn.sigmoid(logit), jnp.float32)


_pool_call = pl.pallas_call(
    _pool_body,
    grid=(N // BNG,),
    in_specs=[
        pl.BlockSpec((BNG, D), lambda n: (n, 0)),
        pl.BlockSpec((1, D), lambda n: (0, 0)),
        pl.BlockSpec((1, 1), lambda n: (0, 0)),
    ],
    out_specs=pl.BlockSpec((1, 1), lambda n: (0, 0)),
    out_shape=jax.ShapeDtypeStruct((1, 1), jnp.float32),
    scratch_shapes=[pltpu.VMEM((1, D), jnp.float32)],
)


def kernel(x, edge_index, edge_types, W_e, b_e, w_ih, w_hh, b_ih, b_hh,
           W_cls, b_cls):
    src = edge_index[0]
    dst = edge_index[1]
    gflat = (edge_types * N + src).astype(jnp.int32).reshape(NW, EPW)
    dflat = dst.astype(jnp.int32).reshape(NW, EPW)
    nmain = NGROUPS * G * CH
    gidx = gflat[:, :nmain].reshape(NW, NGROUPS, G, CH)
    dst2 = dflat[:, :nmain].reshape(NW, NGROUPS, G, CH)
    gidxt = gflat[:, nmain:].reshape(NW, 1, CTAIL)
    dstt = dflat[:, nmain:].reshape(NW, 1, CTAIL)
    zrows = jnp.zeros((ZR, D), jnp.float32)
    bih2 = b_ih.reshape(1, 3 * D)
    bhh2 = b_hh.reshape(1, 3 * D)

    w_pack = jnp.transpose(W_e, (2, 0, 1)).reshape(D, T * D)
    b_pack = b_e.reshape(1, T * D)
    h = x
    for _ in range(STEPS):
        trans = _trans_call(h, w_pack, b_pack).reshape(T * N, D)
        parts = _edge_call(trans, gidx, dst2, gidxt, dstt, zrows)
        h = _gru_call(parts, h, w_ih, w_hh, bih2, bhh2)
    out = _pool_call(h, W_cls.reshape(1, D), b_cls.reshape(1, 1))
    return out[0, 0]
